# baseline (speedup 1.0000x reference)
import functools

import jax
import jax.numpy as jnp
from jax import lax
from jax.experimental import pallas as pl
from jax.experimental.pallas import tpu as pltpu
from jax.experimental.pallas import tpu_sc as plsc

_L = 16


def _sum_along_sc(x0p, x1p, i0r, i1r, *, num_workers, chunk, nbuf):
    B = i0r.shape[0] * i0r.shape[1]
    D = 32
    b_per_w = B // num_workers
    n_chunks = b_per_w // chunk
    ng = n_chunks // nbuf
    lines = chunk // 4

    mesh = plsc.VectorSubcoreMesh(core_axis_name="c", subcore_axis_name="s")

    scratch = [
        pltpu.VMEM((n_chunks, chunk), jnp.int32),
        pltpu.VMEM((n_chunks, chunk), jnp.int32),
    ]
    scratch += [pltpu.VMEM((chunk, 128), jnp.float32)] * (2 * nbuf)
    scratch += [pltpu.VMEM((lines, 128), jnp.float32)] * nbuf
    scratch += [pltpu.SemaphoreType.DMA] * (2 * nbuf)

    @functools.partial(
        pl.kernel,
        mesh=mesh,
        out_type=jax.ShapeDtypeStruct((B // 4, 128), jnp.float32),
        compiler_params=pltpu.CompilerParams(use_tc_tiling_on_sc=True),
        scratch_types=scratch,
    )
    def k(x0_hbm, x1_hbm, i0_hbm, i1_hbm, out_hbm, *s):
        idx0_v, idx1_v = s[0], s[1]
        g0 = s[2:2 + nbuf]
        g1 = s[2 + nbuf:2 + 2 * nbuf]
        ob = s[2 + 2 * nbuf:2 + 3 * nbuf]
        gsem = s[2 + 3 * nbuf:2 + 4 * nbuf]
        osem = s[2 + 4 * nbuf:2 + 5 * nbuf]

        wid = lax.axis_index("s") * 2 + lax.axis_index("c")
        wrow = wid * n_chunks

        pltpu.sync_copy(i0_hbm.at[pl.ds(wrow, n_chunks)], idx0_v)
        pltpu.sync_copy(i1_hbm.at[pl.ds(wrow, n_chunks)], idx1_v)

        def issue(cc, b):
            pltpu.async_copy(x0_hbm.at[idx0_v.at[cc]], g0[b], gsem[b])
            pltpu.async_copy(x1_hbm.at[idx1_v.at[cc]], g1[b], gsem[b])

        def process(cc, b, wait_out):
            pltpu.make_async_copy(x0_hbm.at[idx0_v.at[cc]], g0[b],
                                  gsem[b]).wait()
            pltpu.make_async_copy(x1_hbm.at[idx1_v.at[cc]], g1[b],
                                  gsem[b]).wait()
            if wait_out:
                pltpu.make_async_copy(
                    ob[b], out_hbm.at[pl.ds(0, lines)], osem[b]).wait()

            def add_body(l, _):
                for q in range(4):
                    r = 4 * l + q
                    for v in range(D // _L):
                        src = pl.ds(v * _L, _L)
                        dst = pl.ds(q * D + v * _L, _L)
                        ob[b][l, dst] = g0[b][r, src] + g1[b][r, src]
                return ()

            lax.fori_loop(0, lines, add_body, (), unroll=2)
            pltpu.async_copy(
                ob[b], out_hbm.at[pl.ds((wrow + cc) * lines, lines)], osem[b])

        for b in range(nbuf):
            issue(b, b)
        for b in range(nbuf):
            process(b, b, wait_out=False)
            issue(b + nbuf, b)

        def turn(g, _):
            for b in range(nbuf):
                cc = g * nbuf + b
                process(cc, b, wait_out=True)
                issue(cc + nbuf, b)
            return ()

        lax.fori_loop(1, ng - 1, turn, ())

        for b in range(nbuf):
            process((ng - 1) * nbuf + b, b, wait_out=True)
        for b in range(nbuf):
            pltpu.make_async_copy(
                ob[b], out_hbm.at[pl.ds(0, lines)], osem[b]).wait()

    out = k(x0p, x1p, i0r, i1r)
    return out.reshape(B, D)


def kernel(x0, x1, i0, i1):
    x0p = jnp.pad(x0, ((0, 0), (0, 96)))
    x1p = jnp.pad(x1, ((0, 0), (0, 96)))
    num_workers, chunk = 32, 128
    n_rows = i0.shape[0] * i0.shape[1] // chunk
    i0r = i0.astype(jnp.int32).reshape(n_rows, chunk)
    i1r = i1.astype(jnp.int32).reshape(n_rows, chunk)
    return _sum_along_sc(x0p, x1p, i0r, i1r,
                         num_workers=num_workers, chunk=chunk, nbuf=2)


# speedup vs baseline: 1.1323x; 1.1323x over previous
import functools

import jax
import jax.numpy as jnp
from jax import lax
from jax.experimental import pallas as pl
from jax.experimental.pallas import tpu as pltpu
from jax.experimental.pallas import tpu_sc as plsc

_L = 16


def _sum_along_sc(xc, i0r, i1r, *, num_workers, chunk, nbuf):
    B = i0r.shape[0] * i0r.shape[1]
    D = 32
    b_per_w = B // num_workers
    n_chunks = b_per_w // chunk
    ng = n_chunks // nbuf

    mesh = plsc.VectorSubcoreMesh(core_axis_name="c", subcore_axis_name="s")

    scratch = [
        pltpu.VMEM((n_chunks, chunk), jnp.int32),
        pltpu.VMEM((n_chunks, chunk), jnp.int32),
    ]
    scratch += [pltpu.VMEM((chunk, 128), jnp.float32)] * (2 * nbuf)
    scratch += [pltpu.VMEM((chunk, D), jnp.float32)] * nbuf
    scratch += [pltpu.SemaphoreType.DMA] * (2 * nbuf)

    @functools.partial(
        pl.kernel,
        mesh=mesh,
        out_type=jax.ShapeDtypeStruct((B, D), jnp.float32),
        compiler_params=pltpu.CompilerParams(use_tc_tiling_on_sc=True),
        scratch_types=scratch,
    )
    def k(xc_hbm, i0_hbm, i1_hbm, out_hbm, *s):
        idx0_v, idx1_v = s[0], s[1]
        g0 = s[2:2 + nbuf]
        g1 = s[2 + nbuf:2 + 2 * nbuf]
        ob = s[2 + 2 * nbuf:2 + 3 * nbuf]
        gsem = s[2 + 3 * nbuf:2 + 4 * nbuf]
        osem = s[2 + 4 * nbuf:2 + 5 * nbuf]

        wid = lax.axis_index("s") * 2 + lax.axis_index("c")
        wrow = wid * n_chunks

        pltpu.sync_copy(i0_hbm.at[pl.ds(wrow, n_chunks)], idx0_v)
        pltpu.sync_copy(i1_hbm.at[pl.ds(wrow, n_chunks)], idx1_v)

        def issue(cc, b):
            pltpu.async_copy(xc_hbm.at[idx0_v.at[cc]], g0[b], gsem[b])
            pltpu.async_copy(xc_hbm.at[idx1_v.at[cc]], g1[b], gsem[b])

        def process(cc, b, wait_out):
            pltpu.make_async_copy(xc_hbm.at[idx0_v.at[cc]], g0[b],
                                  gsem[b]).wait()
            pltpu.make_async_copy(xc_hbm.at[idx1_v.at[cc]], g1[b],
                                  gsem[b]).wait()
            if wait_out:
                pltpu.make_async_copy(
                    ob[b], out_hbm.at[pl.ds(0, chunk)], osem[b]).wait()

            def add_body(r, _):
                for v in range(D // _L):
                    dst = pl.ds(v * _L, _L)
                    s1 = pl.ds(D + v * _L, _L)
                    ob[b][r, dst] = g0[b][r, dst] + g1[b][r, s1]
                return ()

            lax.fori_loop(0, chunk, add_body, (), unroll=4)
            pltpu.async_copy(
                ob[b], out_hbm.at[pl.ds((wrow + cc) * chunk, chunk)], osem[b])

        for b in range(nbuf):
            issue(b, b)
        for b in range(nbuf):
            process(b, b, wait_out=False)
            issue(b + nbuf, b)

        def turn(g, _):
            for b in range(nbuf):
                cc = g * nbuf + b
                process(cc, b, wait_out=True)
                issue(cc + nbuf, b)
            return ()

        lax.fori_loop(1, ng - 1, turn, ())

        for b in range(nbuf):
            process((ng - 1) * nbuf + b, b, wait_out=True)
        for b in range(nbuf):
            pltpu.make_async_copy(
                ob[b], out_hbm.at[pl.ds(0, chunk)], osem[b]).wait()

    return k(xc, i0r, i1r)


def kernel(x0, x1, i0, i1):
    z = jnp.zeros((x0.shape[0], 64), jnp.float32)
    xc = jnp.concatenate([x0, x1, z], axis=1)
    num_workers, chunk = 32, 128
    n_rows = i0.shape[0] * i0.shape[1] // chunk
    i0r = i0.astype(jnp.int32).reshape(n_rows, chunk)
    i1r = i1.astype(jnp.int32).reshape(n_rows, chunk)
    return _sum_along_sc(xc, i0r, i1r,
                         num_workers=num_workers, chunk=chunk, nbuf=2)
